# Initial kernel scaffold; baseline (speedup 1.0000x reference)
import jax
import jax.numpy as jnp
from jax.experimental import pallas as pl


def kernel(users_feat, bundles_feat, items_feat, ui_edges, ub_edges, bi_edges):
    raise NotImplementedError("write your pallas kernel here")


# speedup vs baseline: 13.5791x; 13.5791x over previous
import functools

import jax
import jax.numpy as jnp
from jax import lax
from jax.experimental import pallas as pl
from jax.experimental.pallas import tpu as pltpu
from jax.experimental.pallas import tpu_sc as plsc

U = 50000
B = 20000
I = 50000
D = 64

NC = 2
NS = 16
NW = NC * NS
LN = 16
SA = 2048
FLUSH = 1024
STAGE = FLUSH + 32
SG = 128


def _mesh():
    return plsc.VectorSubcoreMesh(core_axis_name="c", subcore_axis_name="s")


def _worker_id():
    return lax.axis_index("s") * NC + lax.axis_index("c")


def _round_up(x, m):
    return (x + m - 1) // m * m


def _al(x):
    return pl.multiple_of(x, 8)


@functools.lru_cache(maxsize=None)
def _make_bucketize(CH, nA, N, span, NB, CAP, symmetric):
    n_full = CH // SA
    tail = CH % SA

    def dir_buckets(lo, hi):
        return [b for b in range(NB) if b * span < hi and (b + 1) * span > lo]

    if symmetric:
        dir_cfg = [(0, dir_buckets(0, nA)), (1, dir_buckets(nA, N))]
    else:
        dir_cfg = [(0, dir_buckets(0, N))]

    def body(rows_hbm, cols_hbm, bsrc_hbm, bdst_hbm, counts_hbm,
             rows_v, cols_v, stg_src, stg_dst, cvec_v):
        w = _worker_id()
        base = w * CH
        lanes = lax.iota(jnp.int32, LN)

        def append(b, cnt, off, src_vals, dst_vals, m):
            sb = b * STAGE
            inc = jnp.where(m, 1, 0)
            excl = plsc.cumsum(inc) - inc
            idx = jnp.where(m, sb + cnt + excl, NB * STAGE)
            plsc.store_scatter(stg_src, [idx], src_vals)
            plsc.store_scatter(stg_dst, [idx], dst_vals)
            cnt = cnt + jnp.sum(inc)

            def do_flush(args):
                cnt, off = args
                lb = (w * NB + b) * CAP
                pltpu.sync_copy(stg_src.at[pl.ds(sb, FLUSH)],
                                bsrc_hbm.at[pl.ds(_al(lb + off), FLUSH)])
                pltpu.sync_copy(stg_dst.at[pl.ds(sb, FLUSH)],
                                bdst_hbm.at[pl.ds(_al(lb + off), FLUSH)])
                rs = stg_src[pl.ds(sb + FLUSH, LN)]
                rd = stg_dst[pl.ds(sb + FLUSH, LN)]
                stg_src[pl.ds(sb, LN)] = rs
                stg_dst[pl.ds(sb, LN)] = rd
                return cnt - FLUSH, off + FLUSH

            return lax.cond(cnt >= FLUSH, do_flush, lambda a: a, (cnt, off))

        def proc_vreg(r, c, m_valid, state):
            outs = list(state)
            if symmetric:
                cpn = c + nA
                pairs = [(r, cpn, dir_cfg[0][1]), (cpn, r, dir_cfg[1][1])]
            else:
                pairs = [(r, c, dir_cfg[0][1])]
            for dval, sval, blist in pairs:
                for b in blist:
                    m = (dval >= b * span) & (dval < (b + 1) * span)
                    if m_valid is not None:
                        m = m & m_valid
                    cnt, off = outs[b]
                    outs[b] = append(b, cnt, off, sval, dval - b * span, m)
            return tuple(outs)

        def run_chunk(buf_len, state):
            nv = buf_len // LN
            rem = buf_len % LN

            def vbody(v, st):
                r = rows_v[pl.ds(v * LN, LN)]
                c = cols_v[pl.ds(v * LN, LN)]
                return proc_vreg(r, c, None, st)

            state = lax.fori_loop(0, nv, vbody, state)
            if rem:
                r = rows_v[pl.ds(nv * LN, LN)]
                c = cols_v[pl.ds(nv * LN, LN)]
                state = proc_vreg(r, c, lanes < rem, state)
            return state

        state = tuple((jnp.int32(0), jnp.int32(0)) for _ in range(NB))

        def outer(i, st):
            pltpu.sync_copy(rows_hbm.at[pl.ds(_al(base + i * SA), SA)], rows_v)
            pltpu.sync_copy(cols_hbm.at[pl.ds(_al(base + i * SA), SA)], cols_v)
            return run_chunk(SA, st)

        state = lax.fori_loop(0, n_full, outer, state)
        if tail:
            pltpu.sync_copy(rows_hbm.at[pl.ds(_al(base + n_full * SA), tail)],
                            rows_v.at[pl.ds(0, tail)])
            pltpu.sync_copy(cols_hbm.at[pl.ds(_al(base + n_full * SA), tail)],
                            cols_v.at[pl.ds(0, tail)])
            state = run_chunk(tail, state)

        cnt_vec = jnp.zeros((LN,), jnp.int32)
        for b in range(NB):
            cnt, off = state[b]
            padded = (cnt + SG - 1) // SG * SG
            neg1 = jnp.full((LN,), -1, jnp.int32)
            sb = b * STAGE
            for k in range(SG // LN):
                pos = sb + cnt + k * LN

                @pl.when(cnt + k * LN < padded)
                def _(pos=pos):
                    stg_src[pl.ds(pos, LN)] = neg1
                    stg_dst[pl.ds(pos, LN)] = neg1

            for j in range(FLUSH // SG):

                @pl.when(j * SG < padded)
                def _(b=b, j=j, off=off, sb=sb):
                    lb = (w * NB + b) * CAP
                    pltpu.sync_copy(
                        stg_src.at[pl.ds(sb + j * SG, SG)],
                        bsrc_hbm.at[pl.ds(_al(lb + off + j * SG), SG)])
                    pltpu.sync_copy(
                        stg_dst.at[pl.ds(sb + j * SG, SG)],
                        bdst_hbm.at[pl.ds(_al(lb + off + j * SG), SG)])

            trips = (off + padded) // SG
            cnt_vec = jnp.where(lanes == b, trips, cnt_vec)
        cvec_v[...] = cnt_vec
        pltpu.sync_copy(cvec_v, counts_hbm.at[pl.ds(_al(w * LN), LN)])

    return pl.kernel(
        body,
        out_type=(
            jax.ShapeDtypeStruct((NW * NB * CAP,), jnp.int32),
            jax.ShapeDtypeStruct((NW * NB * CAP,), jnp.int32),
            jax.ShapeDtypeStruct((NW * LN,), jnp.int32),
        ),
        mesh=_mesh(),
        compiler_params=pltpu.CompilerParams(
            needs_layout_passes=False, use_tc_tiling_on_sc=False),
        scratch_types=[
            pltpu.VMEM((SA,), jnp.int32),
            pltpu.VMEM((SA,), jnp.int32),
            pltpu.VMEM((NB * STAGE + LN,), jnp.int32),
            pltpu.VMEM((NB * STAGE + LN,), jnp.int32),
            pltpu.VMEM((LN,), jnp.int32),
        ],
    )


@functools.lru_cache(maxsize=None)
def _make_degree(NB, span_pad, CAP):
    TPT = span_pad // NS
    n_full = TPT // SG
    tl = TPT % SG
    rounds = NB // NC

    def body(bdst_hbm, counts_hbm, deg_hbm, acc, idst, ones_v, zeros_v,
             bounce_v, cvec_v):
        c = lax.axis_index("c")
        s = lax.axis_index("s")
        lanes = lax.iota(jnp.int32, LN)

        def ib(i, _):
            ones_v[pl.ds(i * LN, LN)] = jnp.full((LN,), 1.0, jnp.float32)
            zeros_v[pl.ds(i * LN, LN)] = jnp.zeros((LN,), jnp.float32)
            return 0

        lax.fori_loop(0, SG // LN, ib, 0)

        off0 = s * TPT
        for r in range(rounds):
            b = r * NC + c
            for j in range(n_full):
                pltpu.sync_copy(zeros_v, acc.at[pl.ds(_al(off0 + j * SG), SG)])
            if tl:
                pltpu.sync_copy(zeros_v.at[pl.ds(0, tl)],
                                acc.at[pl.ds(_al(off0 + n_full * SG), tl)])
            plsc.subcore_barrier()
            for wk in range(NW // NS):
                w = s + NS * wk
                pltpu.sync_copy(counts_hbm.at[pl.ds(_al(w * LN), LN)], cvec_v)
                cv = cvec_v[...]
                trips = jnp.max(jnp.where(lanes == b, cv, 0))
                lb = (w * NB + b) * CAP

                def cb(j, _, lb=lb):
                    pltpu.sync_copy(bdst_hbm.at[pl.ds(_al(lb + j * SG), SG)],
                                    idst)
                    pltpu.sync_copy(
                        ones_v,
                        acc.at[plsc.Indices(idst, ignored_value=-1)],
                        add=True)
                    return 0

                lax.fori_loop(0, trips, cb, 0)
            plsc.subcore_barrier()
            for j in range(n_full):
                pltpu.sync_copy(acc.at[pl.ds(_al(off0 + j * SG), SG)],
                                bounce_v)
                pltpu.sync_copy(
                    bounce_v,
                    deg_hbm.at[pl.ds(_al(b * span_pad + off0 + j * SG), SG)])
            if tl:
                pltpu.sync_copy(acc.at[pl.ds(_al(off0 + n_full * SG), tl)],
                                bounce_v.at[pl.ds(0, tl)])
                pltpu.sync_copy(
                    bounce_v.at[pl.ds(0, tl)],
                    deg_hbm.at[pl.ds(_al(b * span_pad + off0 + n_full * SG), tl)])

    return pl.kernel(
        body,
        out_type=jax.ShapeDtypeStruct((NB * span_pad,), jnp.float32),
        mesh=_mesh(),
        compiler_params=pltpu.CompilerParams(
            needs_layout_passes=False, use_tc_tiling_on_sc=False),
        scratch_types=[
            pltpu.VMEM_SHARED((span_pad,), jnp.float32),
            pltpu.VMEM((SG,), jnp.int32),
            pltpu.VMEM((SG,), jnp.float32),
            pltpu.VMEM((SG,), jnp.float32),
            pltpu.VMEM((SG,), jnp.float32),
            pltpu.VMEM((LN,), jnp.int32),
        ],
    )


@functools.lru_cache(maxsize=None)
def _make_spmm(NB, span_pad, CAP):
    TPT = span_pad // NS
    n_full = TPT // SG
    tl = TPT % SG
    rounds = NB // NC

    def body(gfeat, bsrc_hbm, bdst_hbm, counts_hbm, out_hbm,
             acc, isrc, idst, rows_v, zeros_v, cvec_v):
        c = lax.axis_index("c")
        s = lax.axis_index("s")
        lanes = lax.iota(jnp.int32, LN)

        def zb(i, _):
            for k in range(D // LN):
                zeros_v[i, pl.ds(k * LN, LN)] = jnp.zeros((LN,), jnp.float32)
            return 0

        lax.fori_loop(0, SG, zb, 0)

        off0 = s * TPT
        for r in range(rounds):
            b = r * NC + c
            for j in range(n_full):
                pltpu.sync_copy(zeros_v, acc.at[pl.ds(_al(off0 + j * SG), SG), :])
            if tl:
                pltpu.sync_copy(zeros_v.at[pl.ds(0, tl), :],
                                acc.at[pl.ds(_al(off0 + n_full * SG), tl), :])
            plsc.subcore_barrier()
            for wk in range(NW // NS):
                w = s + NS * wk
                pltpu.sync_copy(counts_hbm.at[pl.ds(_al(w * LN), LN)], cvec_v)
                cv = cvec_v[...]
                trips = jnp.max(jnp.where(lanes == b, cv, 0))
                lb = (w * NB + b) * CAP

                def cb(j, _, lb=lb):
                    pltpu.sync_copy(bsrc_hbm.at[pl.ds(_al(lb + j * SG), SG)],
                                    isrc)
                    pltpu.sync_copy(bdst_hbm.at[pl.ds(_al(lb + j * SG), SG)],
                                    idst)
                    pltpu.sync_copy(
                        gfeat.at[plsc.Indices(isrc, ignored_value=-1)],
                        rows_v)
                    pltpu.sync_copy(
                        rows_v,
                        acc.at[plsc.Indices(idst, ignored_value=-1)],
                        add=True)
                    return 0

                lax.fori_loop(0, trips, cb, 0)
            plsc.subcore_barrier()
            for j in range(n_full):
                pltpu.sync_copy(acc.at[pl.ds(_al(off0 + j * SG), SG), :],
                                rows_v)
                pltpu.sync_copy(
                    rows_v,
                    out_hbm.at[pl.ds(_al(b * span_pad + off0 + j * SG), SG), :])
            if tl:
                pltpu.sync_copy(acc.at[pl.ds(_al(off0 + n_full * SG), tl), :],
                                rows_v.at[pl.ds(0, tl), :])
                pltpu.sync_copy(
                    rows_v.at[pl.ds(0, tl), :],
                    out_hbm.at[pl.ds(_al(b * span_pad + off0 + n_full * SG), tl), :])

    return pl.kernel(
        body,
        out_type=jax.ShapeDtypeStruct((NB * span_pad, D), jnp.float32),
        mesh=_mesh(),
        compiler_params=pltpu.CompilerParams(
            needs_layout_passes=False, use_tc_tiling_on_sc=False),
        scratch_types=[
            pltpu.VMEM_SHARED((span_pad, D), jnp.float32),
            pltpu.VMEM((SG,), jnp.int32),
            pltpu.VMEM((SG,), jnp.int32),
            pltpu.VMEM((SG, D), jnp.float32),
            pltpu.VMEM((SG, D), jnp.float32),
            pltpu.VMEM((LN,), jnp.int32),
        ],
    )


def _unpad(x_pad, NB, span_pad, span, n):
    parts = [x_pad[b * span_pad:b * span_pad + span] for b in range(NB)]
    return jnp.concatenate(parts, axis=0)[:n]


def _l2norm(x):
    norm = jnp.sqrt(jnp.sum(x * x, axis=1, keepdims=True))
    return x / jnp.maximum(norm, 1e-12)


def _graph_setup(e_rows, e_cols, E, nA, N, span, NB, symmetric):
    CH = _round_up((E + NW - 1) // NW, 8)
    if symmetric:
        cap_edges = max(
            2 * CH if any(
                b * span < nA < (b + 1) * span for b in range(NB)) else CH,
            CH)
    else:
        cap_edges = CH
    CAP = _round_up(cap_edges + SG, SG)
    span_pad = _round_up(span, SG)

    pad = NW * CH - E
    if pad:
        fill = jnp.full((pad,), -(nA + 1), jnp.int32)
        e_rows = jnp.concatenate([e_rows, fill])
        e_cols = jnp.concatenate([e_cols, fill])

    bk = _make_bucketize(CH, nA, N, span, NB, CAP, symmetric)
    bsrc, bdst, counts = bk(e_rows, e_cols)
    deg_pad = _make_degree(NB, span_pad, CAP)(bdst, counts)
    deg = _unpad(deg_pad, NB, span_pad, span, N)
    return (bsrc, bdst, counts, CAP, span_pad), deg


def _spmm(gfeat, lists, NB, span_pad, span, N):
    bsrc, bdst, counts, CAP, _ = lists
    out_pad = _make_spmm(NB, span_pad, CAP)(gfeat, bsrc, bdst, counts)
    parts = [out_pad[b * span_pad:b * span_pad + span] for b in range(NB)]
    return jnp.concatenate(parts, axis=0)[:N]


def _propagate(A_feat, B_feat, e_rows, e_cols, E, span, NB, num_layers):
    nA = A_feat.shape[0]
    N = nA + B_feat.shape[0]
    lists, deg = _graph_setup(e_rows, e_cols, E, nA, N, span, NB, True)
    span_pad = lists[4]
    dinv = (1.0 / (jnp.sqrt(deg) + 1e-8))[:, None]

    features = jnp.concatenate([A_feat, B_feat], axis=0)
    total = features
    for i in range(num_layers):
        g = features * dinv
        ssum = _spmm(g, lists, NB, span_pad, span, N)
        features = ssum * dinv / (i + 2)
        total = total + _l2norm(features)
    return total[:nA], total[nA:]


def kernel(users_feat, bundles_feat, items_feat, ui_edges, ub_edges,
           bi_edges):
    IL_users, IL_items = _propagate(
        users_feat, items_feat, ui_edges[0], ui_edges[1],
        E=ui_edges.shape[1], span=25000, NB=4, num_layers=2)

    lists_bi, size = _graph_setup(
        bi_edges[0], bi_edges[1], bi_edges.shape[1], nA=0, N=B,
        span=10000, NB=2, symmetric=False)
    span_pad_bi = lists_bi[4]
    ssum = _spmm(IL_items, lists_bi, 2, span_pad_bi, 10000, B)
    IL_bundles = ssum / (size + 1e-8)[:, None]

    BL_users, BL_bundles = _propagate(
        users_feat, bundles_feat, ub_edges[0], ub_edges[1],
        E=ub_edges.shape[1], span=17500, NB=4, num_layers=2)

    users_out = jnp.concatenate([IL_users, BL_users], axis=1)
    bundles_out = jnp.concatenate([IL_bundles, BL_bundles], axis=1)
    return jnp.concatenate([users_out, bundles_out], axis=0)


# speedup vs baseline: 24.0305x; 1.7697x over previous
import functools

import jax
import jax.numpy as jnp
from jax import lax
from jax.experimental import pallas as pl
from jax.experimental.pallas import tpu as pltpu
from jax.experimental.pallas import tpu_sc as plsc

U = 50000
B = 20000
I = 50000
D = 64

NC = 2
NS = 16
NW = NC * NS
LN = 16
SA = 2048
FLUSH = 1024
STAGE = FLUSH + 32
SG = 128
K = 4


def _mesh():
    return plsc.VectorSubcoreMesh(core_axis_name="c", subcore_axis_name="s")


def _worker_id():
    return lax.axis_index("s") * NC + lax.axis_index("c")


def _round_up(x, m):
    return (x + m - 1) // m * m


def _al(x):
    return pl.multiple_of(x, 8)


@functools.lru_cache(maxsize=None)
def _make_bucketize(CH, nA, N, span, NB, CAP, symmetric):
    n_full = CH // SA
    tail = CH % SA

    def dir_buckets(lo, hi):
        return [b for b in range(NB) if b * span < hi and (b + 1) * span > lo]

    if symmetric:
        dir_cfg = [(0, dir_buckets(0, nA)), (1, dir_buckets(nA, N))]
    else:
        dir_cfg = [(0, dir_buckets(0, N))]

    def body(rows_hbm, cols_hbm, bsrc_hbm, bdst_hbm, counts_hbm,
             rows_v, cols_v, stg_src, stg_dst, cvec_v):
        w = _worker_id()
        base = w * CH
        lanes = lax.iota(jnp.int32, LN)

        def append(b, cnt, off, src_vals, dst_vals, m):
            sb = b * STAGE
            inc = jnp.where(m, 1, 0)
            excl = plsc.cumsum(inc) - inc
            idx = jnp.where(m, sb + cnt + excl, NB * STAGE)
            plsc.store_scatter(stg_src, [idx], src_vals)
            plsc.store_scatter(stg_dst, [idx], dst_vals)
            cnt = cnt + jnp.sum(inc)

            def do_flush(args):
                cnt, off = args
                lb = (w * NB + b) * CAP
                pltpu.sync_copy(stg_src.at[pl.ds(sb, FLUSH)],
                                bsrc_hbm.at[pl.ds(_al(lb + off), FLUSH)])
                pltpu.sync_copy(stg_dst.at[pl.ds(sb, FLUSH)],
                                bdst_hbm.at[pl.ds(_al(lb + off), FLUSH)])
                rs = stg_src[pl.ds(sb + FLUSH, LN)]
                rd = stg_dst[pl.ds(sb + FLUSH, LN)]
                stg_src[pl.ds(sb, LN)] = rs
                stg_dst[pl.ds(sb, LN)] = rd
                return cnt - FLUSH, off + FLUSH

            return lax.cond(cnt >= FLUSH, do_flush, lambda a: a, (cnt, off))

        def proc_vreg(r, c, m_valid, state):
            outs = list(state)
            if symmetric:
                cpn = c + nA
                pairs = [(r, cpn, dir_cfg[0][1]), (cpn, r, dir_cfg[1][1])]
            else:
                pairs = [(r, c, dir_cfg[0][1])]
            for dval, sval, blist in pairs:
                for b in blist:
                    m = (dval >= b * span) & (dval < (b + 1) * span)
                    if m_valid is not None:
                        m = m & m_valid
                    cnt, off = outs[b]
                    outs[b] = append(b, cnt, off, sval, dval - b * span, m)
            return tuple(outs)

        def run_chunk(buf_len, state):
            nv = buf_len // LN
            rem = buf_len % LN

            def vbody(v, st):
                r = rows_v[pl.ds(v * LN, LN)]
                c = cols_v[pl.ds(v * LN, LN)]
                return proc_vreg(r, c, None, st)

            state = lax.fori_loop(0, nv, vbody, state)
            if rem:
                r = rows_v[pl.ds(nv * LN, LN)]
                c = cols_v[pl.ds(nv * LN, LN)]
                state = proc_vreg(r, c, lanes < rem, state)
            return state

        state = tuple((jnp.int32(0), jnp.int32(0)) for _ in range(NB))

        def outer(i, st):
            pltpu.sync_copy(rows_hbm.at[pl.ds(_al(base + i * SA), SA)], rows_v)
            pltpu.sync_copy(cols_hbm.at[pl.ds(_al(base + i * SA), SA)], cols_v)
            return run_chunk(SA, st)

        state = lax.fori_loop(0, n_full, outer, state)
        if tail:
            pltpu.sync_copy(rows_hbm.at[pl.ds(_al(base + n_full * SA), tail)],
                            rows_v.at[pl.ds(0, tail)])
            pltpu.sync_copy(cols_hbm.at[pl.ds(_al(base + n_full * SA), tail)],
                            cols_v.at[pl.ds(0, tail)])
            state = run_chunk(tail, state)

        cnt_vec = jnp.zeros((LN,), jnp.int32)
        for b in range(NB):
            cnt, off = state[b]
            padded = (cnt + SG - 1) // SG * SG
            neg1 = jnp.full((LN,), -1, jnp.int32)
            sb = b * STAGE
            for k in range(SG // LN):
                pos = sb + cnt + k * LN

                @pl.when(cnt + k * LN < padded)
                def _(pos=pos):
                    stg_src[pl.ds(pos, LN)] = neg1
                    stg_dst[pl.ds(pos, LN)] = neg1

            for j in range(FLUSH // SG):

                @pl.when(j * SG < padded)
                def _(b=b, j=j, off=off, sb=sb):
                    lb = (w * NB + b) * CAP
                    pltpu.sync_copy(
                        stg_src.at[pl.ds(sb + j * SG, SG)],
                        bsrc_hbm.at[pl.ds(_al(lb + off + j * SG), SG)])
                    pltpu.sync_copy(
                        stg_dst.at[pl.ds(sb + j * SG, SG)],
                        bdst_hbm.at[pl.ds(_al(lb + off + j * SG), SG)])

            trips = (off + padded) // SG
            cnt_vec = jnp.where(lanes == b, trips, cnt_vec)
        cvec_v[...] = cnt_vec
        pltpu.sync_copy(cvec_v, counts_hbm.at[pl.ds(_al(w * LN), LN)])

    return pl.kernel(
        body,
        out_type=(
            jax.ShapeDtypeStruct((NW * NB * CAP,), jnp.int32),
            jax.ShapeDtypeStruct((NW * NB * CAP,), jnp.int32),
            jax.ShapeDtypeStruct((NW * LN,), jnp.int32),
        ),
        mesh=_mesh(),
        compiler_params=pltpu.CompilerParams(
            needs_layout_passes=False, use_tc_tiling_on_sc=False),
        scratch_types=[
            pltpu.VMEM((SA,), jnp.int32),
            pltpu.VMEM((SA,), jnp.int32),
            pltpu.VMEM((NB * STAGE + LN,), jnp.int32),
            pltpu.VMEM((NB * STAGE + LN,), jnp.int32),
            pltpu.VMEM((LN,), jnp.int32),
        ],
    )


@functools.lru_cache(maxsize=None)
def _make_degree(NB, span_pad, CAP):
    TPT = span_pad // NS
    n_full = TPT // SG
    tl = TPT % SG
    rounds = NB // NC

    def body(bdst_hbm, counts_hbm, deg_hbm, acc, idst0, idst1, idst2, idst3,
             ones_v, zeros_v, bounce_v, cvec_v, sem):
        c = lax.axis_index("c")
        s = lax.axis_index("s")
        lanes = lax.iota(jnp.int32, LN)
        idst = [idst0, idst1, idst2, idst3]

        def ib(i, _):
            ones_v[pl.ds(i * LN, LN)] = jnp.full((LN,), 1.0, jnp.float32)
            zeros_v[pl.ds(i * LN, LN)] = jnp.zeros((LN,), jnp.float32)
            return 0

        lax.fori_loop(0, SG // LN, ib, 0)

        off0 = s * TPT
        for r in range(rounds):
            b = r * NC + c
            for j0 in range(0, n_full, K):
                hs = [pltpu.async_copy(
                    zeros_v, acc.at[pl.ds(_al(off0 + j * SG), SG)], sem)
                    for j in range(j0, min(j0 + K, n_full))]
                for h in hs:
                    h.wait()
            if tl:
                pltpu.sync_copy(zeros_v.at[pl.ds(0, tl)],
                                acc.at[pl.ds(_al(off0 + n_full * SG), tl)])
            plsc.subcore_barrier()
            for wk in range(NW // NS):
                w = s + NS * wk
                pltpu.sync_copy(counts_hbm.at[pl.ds(_al(w * LN), LN)], cvec_v)
                cv = cvec_v[...]
                trips = jnp.max(jnp.where(lanes == b, cv, 0))
                full = trips // K
                rem = trips - full * K
                lb = (w * NB + b) * CAP

                def gb(g, _, lb=lb):
                    hs = [pltpu.async_copy(
                        bdst_hbm.at[pl.ds(_al(lb + (g * K + u) * SG), SG)],
                        idst[u], sem) for u in range(K)]
                    for h in hs:
                        h.wait()
                    ss = [pltpu.async_copy(
                        ones_v,
                        acc.at[plsc.Indices(idst[u], ignored_value=-1)],
                        sem, add=True) for u in range(K)]
                    for h in ss:
                        h.wait()
                    return 0

                lax.fori_loop(0, full, gb, 0)
                for u in range(K - 1):

                    @pl.when(rem > u)
                    def _(u=u, lb=lb, full=full):
                        pltpu.sync_copy(
                            bdst_hbm.at[pl.ds(_al(lb + (full * K + u) * SG),
                                              SG)],
                            idst[u])
                        pltpu.sync_copy(
                            ones_v,
                            acc.at[plsc.Indices(idst[u], ignored_value=-1)],
                            add=True)
            plsc.subcore_barrier()
            for j in range(n_full):
                pltpu.sync_copy(acc.at[pl.ds(_al(off0 + j * SG), SG)],
                                bounce_v)
                pltpu.sync_copy(
                    bounce_v,
                    deg_hbm.at[pl.ds(_al(b * span_pad + off0 + j * SG), SG)])
            if tl:
                pltpu.sync_copy(acc.at[pl.ds(_al(off0 + n_full * SG), tl)],
                                bounce_v.at[pl.ds(0, tl)])
                pltpu.sync_copy(
                    bounce_v.at[pl.ds(0, tl)],
                    deg_hbm.at[pl.ds(_al(b * span_pad + off0 + n_full * SG), tl)])

    return pl.kernel(
        body,
        out_type=jax.ShapeDtypeStruct((NB * span_pad,), jnp.float32),
        mesh=_mesh(),
        compiler_params=pltpu.CompilerParams(
            needs_layout_passes=False, use_tc_tiling_on_sc=False),
        scratch_types=[
            pltpu.VMEM_SHARED((span_pad,), jnp.float32),
            pltpu.VMEM((SG,), jnp.int32),
            pltpu.VMEM((SG,), jnp.int32),
            pltpu.VMEM((SG,), jnp.int32),
            pltpu.VMEM((SG,), jnp.int32),
            pltpu.VMEM((SG,), jnp.float32),
            pltpu.VMEM((SG,), jnp.float32),
            pltpu.VMEM((SG,), jnp.float32),
            pltpu.VMEM((LN,), jnp.int32),
            pltpu.SemaphoreType.DMA,
        ],
    )


@functools.lru_cache(maxsize=None)
def _make_spmm(NB, span_pad, CAP):
    TPT = span_pad // NS
    n_full = TPT // SG
    tl = TPT % SG
    rounds = NB // NC
    per_tile = (2 * 1024 * 1024 - 1 - span_pad * D) // NS
    KE = max(1, min(K, (per_tile - 2048) // (SG * D)))

    def body(*refs):
        gfeat, bsrc_hbm, bdst_hbm, counts_hbm, out_hbm, acc = refs[:6]
        isrc = list(refs[6:6 + KE])
        idst = list(refs[6 + KE:6 + 2 * KE])
        rows = list(refs[6 + 2 * KE:6 + 3 * KE])
        cvec_v = refs[6 + 3 * KE]
        sem = refs[6 + 3 * KE + 1]
        c = lax.axis_index("c")
        s = lax.axis_index("s")
        lanes = lax.iota(jnp.int32, LN)

        off0 = s * TPT
        for r in range(rounds):
            b = r * NC + c

            def zb(i, _):
                for k in range(D // LN):
                    rows[0][i, pl.ds(k * LN, LN)] = jnp.zeros((LN,),
                                                              jnp.float32)
                return 0

            lax.fori_loop(0, SG, zb, 0)
            for j0 in range(0, n_full, K):
                hs = [pltpu.async_copy(
                    rows[0], acc.at[pl.ds(_al(off0 + j * SG), SG), :], sem)
                    for j in range(j0, min(j0 + K, n_full))]
                for h in hs:
                    h.wait()
            if tl:
                pltpu.sync_copy(rows[0].at[pl.ds(0, tl), :],
                                acc.at[pl.ds(_al(off0 + n_full * SG), tl), :])
            plsc.subcore_barrier()
            for wk in range(NW // NS):
                w = s + NS * wk
                pltpu.sync_copy(counts_hbm.at[pl.ds(_al(w * LN), LN)], cvec_v)
                cv = cvec_v[...]
                trips = jnp.max(jnp.where(lanes == b, cv, 0))
                full = trips // KE
                rem = trips - full * KE
                lb = (w * NB + b) * CAP

                def gb(g, _, lb=lb):
                    hs = []
                    for u in range(KE):
                        off = _al(lb + (g * KE + u) * SG)
                        hs.append(pltpu.async_copy(
                            bsrc_hbm.at[pl.ds(off, SG)], isrc[u], sem))
                        hs.append(pltpu.async_copy(
                            bdst_hbm.at[pl.ds(off, SG)], idst[u], sem))
                    for h in hs:
                        h.wait()
                    gs = [pltpu.async_copy(
                        gfeat.at[plsc.Indices(isrc[u], ignored_value=-1)],
                        rows[u], sem) for u in range(KE)]
                    for h in gs:
                        h.wait()
                    ss = [pltpu.async_copy(
                        rows[u],
                        acc.at[plsc.Indices(idst[u], ignored_value=-1)],
                        sem, add=True) for u in range(KE)]
                    for h in ss:
                        h.wait()
                    return 0

                lax.fori_loop(0, full, gb, 0)
                for u in range(KE - 1):

                    @pl.when(rem > u)
                    def _(u=u, lb=lb, full=full):
                        off = _al(lb + (full * KE + u) * SG)
                        pltpu.sync_copy(bsrc_hbm.at[pl.ds(off, SG)], isrc[u])
                        pltpu.sync_copy(bdst_hbm.at[pl.ds(off, SG)], idst[u])
                        pltpu.sync_copy(
                            gfeat.at[plsc.Indices(isrc[u], ignored_value=-1)],
                            rows[u])
                        pltpu.sync_copy(
                            rows[u],
                            acc.at[plsc.Indices(idst[u], ignored_value=-1)],
                            add=True)
            plsc.subcore_barrier()
            for j0 in range(0, n_full, KE):
                js = range(j0, min(j0 + KE, n_full))
                hs = [pltpu.async_copy(
                    acc.at[pl.ds(_al(off0 + j * SG), SG), :],
                    rows[j - j0], sem) for j in js]
                for h in hs:
                    h.wait()
                hs = [pltpu.async_copy(
                    rows[j - j0],
                    out_hbm.at[pl.ds(_al(b * span_pad + off0 + j * SG), SG), :],
                    sem) for j in js]
                for h in hs:
                    h.wait()
            if tl:
                pltpu.sync_copy(acc.at[pl.ds(_al(off0 + n_full * SG), tl), :],
                                rows[0].at[pl.ds(0, tl), :])
                pltpu.sync_copy(
                    rows[0].at[pl.ds(0, tl), :],
                    out_hbm.at[pl.ds(_al(b * span_pad + off0 + n_full * SG), tl), :])

    return pl.kernel(
        body,
        out_type=jax.ShapeDtypeStruct((NB * span_pad, D), jnp.float32),
        mesh=_mesh(),
        compiler_params=pltpu.CompilerParams(
            needs_layout_passes=False, use_tc_tiling_on_sc=False),
        scratch_types=(
            [pltpu.VMEM_SHARED((span_pad, D), jnp.float32)]
            + [pltpu.VMEM((SG,), jnp.int32) for _ in range(2 * KE)]
            + [pltpu.VMEM((SG, D), jnp.float32) for _ in range(KE)]
            + [pltpu.VMEM((LN,), jnp.int32), pltpu.SemaphoreType.DMA]
        ),
    )


def _unpad(x_pad, NB, span_pad, span, n):
    parts = [x_pad[b * span_pad:b * span_pad + span] for b in range(NB)]
    return jnp.concatenate(parts, axis=0)[:n]


def _l2norm(x):
    norm = jnp.sqrt(jnp.sum(x * x, axis=1, keepdims=True))
    return x / jnp.maximum(norm, 1e-12)


def _graph_setup(e_rows, e_cols, E, nA, N, span, NB, symmetric):
    CH = _round_up((E + NW - 1) // NW, 8)
    if symmetric:
        cap_edges = max(
            2 * CH if any(
                b * span < nA < (b + 1) * span for b in range(NB)) else CH,
            CH)
    else:
        cap_edges = CH
    CAP = _round_up(cap_edges + SG, SG)
    span_pad = _round_up(span, SG)

    pad = NW * CH - E
    if pad:
        fill = jnp.full((pad,), -(nA + 1), jnp.int32)
        e_rows = jnp.concatenate([e_rows, fill])
        e_cols = jnp.concatenate([e_cols, fill])

    bk = _make_bucketize(CH, nA, N, span, NB, CAP, symmetric)
    bsrc, bdst, counts = bk(e_rows, e_cols)
    deg_pad = _make_degree(NB, span_pad, CAP)(bdst, counts)
    deg = _unpad(deg_pad, NB, span_pad, span, N)
    return (bsrc, bdst, counts, CAP, span_pad), deg


def _spmm(gfeat, lists, NB, span_pad, span, N):
    bsrc, bdst, counts, CAP, _ = lists
    out_pad = _make_spmm(NB, span_pad, CAP)(gfeat, bsrc, bdst, counts)
    parts = [out_pad[b * span_pad:b * span_pad + span] for b in range(NB)]
    return jnp.concatenate(parts, axis=0)[:N]


def _propagate(A_feat, B_feat, e_rows, e_cols, E, span, NB, num_layers):
    nA = A_feat.shape[0]
    N = nA + B_feat.shape[0]
    lists, deg = _graph_setup(e_rows, e_cols, E, nA, N, span, NB, True)
    span_pad = lists[4]
    dinv = (1.0 / (jnp.sqrt(deg) + 1e-8))[:, None]

    features = jnp.concatenate([A_feat, B_feat], axis=0)
    total = features
    for i in range(num_layers):
        g = features * dinv
        ssum = _spmm(g, lists, NB, span_pad, span, N)
        features = ssum * dinv / (i + 2)
        total = total + _l2norm(features)
    return total[:nA], total[nA:]


def kernel(users_feat, bundles_feat, items_feat, ui_edges, ub_edges,
           bi_edges):
    IL_users, IL_items = _propagate(
        users_feat, items_feat, ui_edges[0], ui_edges[1],
        E=ui_edges.shape[1], span=25000, NB=4, num_layers=2)

    lists_bi, size = _graph_setup(
        bi_edges[0], bi_edges[1], bi_edges.shape[1], nA=0, N=B,
        span=10000, NB=2, symmetric=False)
    span_pad_bi = lists_bi[4]
    ssum = _spmm(IL_items, lists_bi, 2, span_pad_bi, 10000, B)
    IL_bundles = ssum / (size + 1e-8)[:, None]

    BL_users, BL_bundles = _propagate(
        users_feat, bundles_feat, ub_edges[0], ub_edges[1],
        E=ub_edges.shape[1], span=17500, NB=4, num_layers=2)

    users_out = jnp.concatenate([IL_users, BL_users], axis=1)
    bundles_out = jnp.concatenate([IL_bundles, BL_bundles], axis=1)
    return jnp.concatenate([users_out, bundles_out], axis=0)


# speedup vs baseline: 24.7113x; 1.0283x over previous
import functools

import jax
import jax.numpy as jnp
from jax import lax
from jax.experimental import pallas as pl
from jax.experimental.pallas import tpu as pltpu
from jax.experimental.pallas import tpu_sc as plsc

U = 50000
B = 20000
I = 50000
D = 64

NC = 2
NS = 16
NW = NC * NS
LN = 16
SA = 2048
FLUSH = 1024
STAGE = FLUSH + 32
SG = 128
K = 4


def _mesh():
    return plsc.VectorSubcoreMesh(core_axis_name="c", subcore_axis_name="s")


def _worker_id():
    return lax.axis_index("s") * NC + lax.axis_index("c")


def _round_up(x, m):
    return (x + m - 1) // m * m


def _al(x):
    return pl.multiple_of(x, 8)


@functools.lru_cache(maxsize=None)
def _make_bucketize(CH, nA, N, span, NB, CAP, symmetric):
    n_full = CH // SA
    tail = CH % SA

    def dir_buckets(lo, hi):
        return [b for b in range(NB) if b * span < hi and (b + 1) * span > lo]

    if symmetric:
        dir_cfg = [(0, dir_buckets(0, nA)), (1, dir_buckets(nA, N))]
    else:
        dir_cfg = [(0, dir_buckets(0, N))]

    def body(rows_hbm, cols_hbm, bsrc_hbm, bdst_hbm, counts_hbm,
             rows_v, cols_v, stg_src, stg_dst, cvec_v):
        w = _worker_id()
        base = w * CH
        lanes = lax.iota(jnp.int32, LN)

        def append(b, cnt, off, src_vals, dst_vals, m):
            sb = b * STAGE
            inc = jnp.where(m, 1, 0)
            excl = plsc.cumsum(inc) - inc
            idx = jnp.where(m, sb + cnt + excl, NB * STAGE)
            plsc.store_scatter(stg_src, [idx], src_vals)
            plsc.store_scatter(stg_dst, [idx], dst_vals)
            cnt = cnt + jnp.sum(inc)

            def do_flush(args):
                cnt, off = args
                lb = (w * NB + b) * CAP
                pltpu.sync_copy(stg_src.at[pl.ds(sb, FLUSH)],
                                bsrc_hbm.at[pl.ds(_al(lb + off), FLUSH)])
                pltpu.sync_copy(stg_dst.at[pl.ds(sb, FLUSH)],
                                bdst_hbm.at[pl.ds(_al(lb + off), FLUSH)])
                rs = stg_src[pl.ds(sb + FLUSH, LN)]
                rd = stg_dst[pl.ds(sb + FLUSH, LN)]
                stg_src[pl.ds(sb, LN)] = rs
                stg_dst[pl.ds(sb, LN)] = rd
                return cnt - FLUSH, off + FLUSH

            return lax.cond(cnt >= FLUSH, do_flush, lambda a: a, (cnt, off))

        def proc_vreg(r, c, m_valid, state):
            outs = list(state)
            if symmetric:
                cpn = c + nA
                pairs = [(r, cpn, dir_cfg[0][1]), (cpn, r, dir_cfg[1][1])]
            else:
                pairs = [(r, c, dir_cfg[0][1])]
            for dval, sval, blist in pairs:
                for b in blist:
                    m = (dval >= b * span) & (dval < (b + 1) * span)
                    if m_valid is not None:
                        m = m & m_valid
                    cnt, off = outs[b]
                    outs[b] = append(b, cnt, off, sval, dval - b * span, m)
            return tuple(outs)

        def run_chunk(buf_len, state):
            nv = buf_len // LN
            rem = buf_len % LN

            def vbody(v, st):
                r = rows_v[pl.ds(v * LN, LN)]
                c = cols_v[pl.ds(v * LN, LN)]
                return proc_vreg(r, c, None, st)

            state = lax.fori_loop(0, nv, vbody, state)
            if rem:
                r = rows_v[pl.ds(nv * LN, LN)]
                c = cols_v[pl.ds(nv * LN, LN)]
                state = proc_vreg(r, c, lanes < rem, state)
            return state

        state = tuple((jnp.int32(0), jnp.int32(0)) for _ in range(NB))

        def outer(i, st):
            pltpu.sync_copy(rows_hbm.at[pl.ds(_al(base + i * SA), SA)], rows_v)
            pltpu.sync_copy(cols_hbm.at[pl.ds(_al(base + i * SA), SA)], cols_v)
            return run_chunk(SA, st)

        state = lax.fori_loop(0, n_full, outer, state)
        if tail:
            pltpu.sync_copy(rows_hbm.at[pl.ds(_al(base + n_full * SA), tail)],
                            rows_v.at[pl.ds(0, tail)])
            pltpu.sync_copy(cols_hbm.at[pl.ds(_al(base + n_full * SA), tail)],
                            cols_v.at[pl.ds(0, tail)])
            state = run_chunk(tail, state)

        cnt_vec = jnp.zeros((LN,), jnp.int32)
        for b in range(NB):
            cnt, off = state[b]
            padded = (cnt + SG - 1) // SG * SG
            neg1 = jnp.full((LN,), -1, jnp.int32)
            sb = b * STAGE
            for k in range(SG // LN):
                pos = sb + cnt + k * LN

                @pl.when(cnt + k * LN < padded)
                def _(pos=pos):
                    stg_src[pl.ds(pos, LN)] = neg1
                    stg_dst[pl.ds(pos, LN)] = neg1

            for j in range(FLUSH // SG):

                @pl.when(j * SG < padded)
                def _(b=b, j=j, off=off, sb=sb):
                    lb = (w * NB + b) * CAP
                    pltpu.sync_copy(
                        stg_src.at[pl.ds(sb + j * SG, SG)],
                        bsrc_hbm.at[pl.ds(_al(lb + off + j * SG), SG)])
                    pltpu.sync_copy(
                        stg_dst.at[pl.ds(sb + j * SG, SG)],
                        bdst_hbm.at[pl.ds(_al(lb + off + j * SG), SG)])

            trips = (off + padded) // SG
            cnt_vec = jnp.where(lanes == b, trips, cnt_vec)
        cvec_v[...] = cnt_vec
        pltpu.sync_copy(cvec_v, counts_hbm.at[pl.ds(_al(w * LN), LN)])

    return pl.kernel(
        body,
        out_type=(
            jax.ShapeDtypeStruct((NW * NB * CAP,), jnp.int32),
            jax.ShapeDtypeStruct((NW * NB * CAP,), jnp.int32),
            jax.ShapeDtypeStruct((NW * LN,), jnp.int32),
        ),
        mesh=_mesh(),
        compiler_params=pltpu.CompilerParams(
            needs_layout_passes=False, use_tc_tiling_on_sc=False),
        scratch_types=[
            pltpu.VMEM((SA,), jnp.int32),
            pltpu.VMEM((SA,), jnp.int32),
            pltpu.VMEM((NB * STAGE + LN,), jnp.int32),
            pltpu.VMEM((NB * STAGE + LN,), jnp.int32),
            pltpu.VMEM((LN,), jnp.int32),
        ],
    )


@functools.lru_cache(maxsize=None)
def _make_degree(NB, span_pad, CAP):
    TPT = span_pad // NS
    n_full = TPT // SG
    tl = TPT % SG
    rounds = NB // NC

    def body(bdst_hbm, counts_hbm, deg_hbm, acc, idst0, idst1, idst2, idst3,
             ones_v, zeros_v, bounce_v, cvec_v, sem):
        c = lax.axis_index("c")
        s = lax.axis_index("s")
        lanes = lax.iota(jnp.int32, LN)
        idst = [idst0, idst1, idst2, idst3]

        def ib(i, _):
            ones_v[pl.ds(i * LN, LN)] = jnp.full((LN,), 1.0, jnp.float32)
            zeros_v[pl.ds(i * LN, LN)] = jnp.zeros((LN,), jnp.float32)
            return 0

        lax.fori_loop(0, SG // LN, ib, 0)

        off0 = s * TPT
        for r in range(rounds):
            b = r * NC + c
            for j0 in range(0, n_full, K):
                hs = [pltpu.async_copy(
                    zeros_v, acc.at[pl.ds(_al(off0 + j * SG), SG)], sem)
                    for j in range(j0, min(j0 + K, n_full))]
                for h in hs:
                    h.wait()
            if tl:
                pltpu.sync_copy(zeros_v.at[pl.ds(0, tl)],
                                acc.at[pl.ds(_al(off0 + n_full * SG), tl)])
            plsc.subcore_barrier()
            for wk in range(NW // NS):
                w = s + NS * wk
                pltpu.sync_copy(counts_hbm.at[pl.ds(_al(w * LN), LN)], cvec_v)
                cv = cvec_v[...]
                trips = jnp.max(jnp.where(lanes == b, cv, 0))
                full = trips // K
                rem = trips - full * K
                lb = (w * NB + b) * CAP

                def gb(g, _, lb=lb):
                    hs = [pltpu.async_copy(
                        bdst_hbm.at[pl.ds(_al(lb + (g * K + u) * SG), SG)],
                        idst[u], sem) for u in range(K)]
                    for h in hs:
                        h.wait()
                    ss = [pltpu.async_copy(
                        ones_v,
                        acc.at[plsc.Indices(idst[u], ignored_value=-1)],
                        sem, add=True) for u in range(K)]
                    for h in ss:
                        h.wait()
                    return 0

                lax.fori_loop(0, full, gb, 0)
                for u in range(K - 1):

                    @pl.when(rem > u)
                    def _(u=u, lb=lb, full=full):
                        pltpu.sync_copy(
                            bdst_hbm.at[pl.ds(_al(lb + (full * K + u) * SG),
                                              SG)],
                            idst[u])
                        pltpu.sync_copy(
                            ones_v,
                            acc.at[plsc.Indices(idst[u], ignored_value=-1)],
                            add=True)
            plsc.subcore_barrier()
            for j in range(n_full):
                pltpu.sync_copy(acc.at[pl.ds(_al(off0 + j * SG), SG)],
                                bounce_v)
                pltpu.sync_copy(
                    bounce_v,
                    deg_hbm.at[pl.ds(_al(b * span_pad + off0 + j * SG), SG)])
            if tl:
                pltpu.sync_copy(acc.at[pl.ds(_al(off0 + n_full * SG), tl)],
                                bounce_v.at[pl.ds(0, tl)])
                pltpu.sync_copy(
                    bounce_v.at[pl.ds(0, tl)],
                    deg_hbm.at[pl.ds(_al(b * span_pad + off0 + n_full * SG), tl)])

    return pl.kernel(
        body,
        out_type=jax.ShapeDtypeStruct((NB * span_pad,), jnp.float32),
        mesh=_mesh(),
        compiler_params=pltpu.CompilerParams(
            needs_layout_passes=False, use_tc_tiling_on_sc=False),
        scratch_types=[
            pltpu.VMEM_SHARED((span_pad,), jnp.float32),
            pltpu.VMEM((SG,), jnp.int32),
            pltpu.VMEM((SG,), jnp.int32),
            pltpu.VMEM((SG,), jnp.int32),
            pltpu.VMEM((SG,), jnp.int32),
            pltpu.VMEM((SG,), jnp.float32),
            pltpu.VMEM((SG,), jnp.float32),
            pltpu.VMEM((SG,), jnp.float32),
            pltpu.VMEM((LN,), jnp.int32),
            pltpu.SemaphoreType.DMA,
        ],
    )


@functools.lru_cache(maxsize=None)
def _make_spmm(NB, span_pad, CAP):
    TPT = span_pad // NS
    n_full = TPT // SG
    tl = TPT % SG
    rounds = NB // NC
    per_tile = (2 * 1024 * 1024 - 1 - span_pad * D) // NS
    KE2 = min(K, (per_tile - 4096) // (2 * SG * D))
    if KE2 >= 2:
        R, KE = 2, KE2
    else:
        R, KE = 1, max(1, min(K, (per_tile - 2048) // (SG * D)))

    def body(*refs):
        gfeat, bsrc_hbm, bdst_hbm, counts_hbm, out_hbm, acc = refs[:6]
        flat = list(refs[6:6 + 3 * R * KE])
        isrc = [flat[p * KE:(p + 1) * KE] for p in range(R)]
        idst = [flat[(R + p) * KE:(R + p + 1) * KE] for p in range(R)]
        rows = [flat[(2 * R + p) * KE:(2 * R + p + 1) * KE]
                for p in range(R)]
        cvec_v = refs[6 + 3 * R * KE]
        sem = refs[6 + 3 * R * KE + 1]
        sem_i = refs[6 + 3 * R * KE + 2]
        sem_s = refs[6 + 3 * R * KE + 3]
        c = lax.axis_index("c")
        s = lax.axis_index("s")
        lanes = lax.iota(jnp.int32, LN)

        off0 = s * TPT
        z0 = rows[0][0]
        rowsf = [rw for p in range(R) for rw in rows[p]]
        for r in range(rounds):
            b = r * NC + c

            def zb(i, _):
                for k in range(D // LN):
                    z0[i, pl.ds(k * LN, LN)] = jnp.zeros((LN,), jnp.float32)
                return 0

            lax.fori_loop(0, SG, zb, 0)
            for j0 in range(0, n_full, K):
                hs = [pltpu.async_copy(
                    z0, acc.at[pl.ds(_al(off0 + j * SG), SG), :], sem)
                    for j in range(j0, min(j0 + K, n_full))]
                for h in hs:
                    h.wait()
            if tl:
                pltpu.sync_copy(z0.at[pl.ds(0, tl), :],
                                acc.at[pl.ds(_al(off0 + n_full * SG), tl), :])
            plsc.subcore_barrier()
            for wk in range(NW // NS):
                w = s + NS * wk
                pltpu.sync_copy(counts_hbm.at[pl.ds(_al(w * LN), LN)], cvec_v)
                cv = cvec_v[...]
                trips = jnp.max(jnp.where(lanes == b, cv, 0))
                full = trips // KE
                rem = trips - full * KE
                lb = (w * NB + b) * CAP

                def fire_idx(g, p, lb=lb):
                    for u in range(KE):
                        off = _al(lb + (g * KE + u) * SG)
                        pltpu.async_copy(bsrc_hbm.at[pl.ds(off, SG)],
                                         isrc[p][u], sem_i)
                        pltpu.async_copy(bdst_hbm.at[pl.ds(off, SG)],
                                         idst[p][u], sem_i)

                def drain_idx(lb=lb):
                    for u in range(KE):
                        pltpu.make_async_copy(bsrc_hbm.at[pl.ds(_al(lb), SG)],
                                              isrc[0][u], sem_i).wait()
                        pltpu.make_async_copy(bsrc_hbm.at[pl.ds(_al(lb), SG)],
                                              idst[0][u], sem_i).wait()

                def drain_sc():
                    for u in range(KE):
                        pltpu.make_async_copy(gfeat.at[pl.ds(0, SG), :],
                                              rows[0][u], sem_s).wait()

                if R == 2:

                    def sub(g, p):
                        q = 1 - p
                        drain_idx()
                        gs = [pltpu.async_copy(
                            gfeat.at[plsc.Indices(isrc[p][u],
                                                  ignored_value=-1)],
                            rows[p][u], sem) for u in range(KE)]

                        @pl.when(g > 0)
                        def _():
                            drain_sc()

                        fire_idx(g + 1, q)
                        for h in gs:
                            h.wait()
                        for u in range(KE):
                            pltpu.async_copy(
                                rows[p][u],
                                acc.at[plsc.Indices(idst[p][u],
                                                    ignored_value=-1)],
                                sem_s, add=True)

                    fire_idx(jnp.int32(0), 0)

                    def gb2(h2, _):
                        sub(2 * h2, 0)
                        sub(2 * h2 + 1, 1)
                        return 0

                    full2 = full // 2
                    lax.fori_loop(0, full2, gb2, 0)

                    @pl.when(full - full2 * 2 > 0)
                    def _(full=full):
                        sub(full - 1, 0)

                    drain_idx()

                    @pl.when(full > 0)
                    def _():
                        drain_sc()
                else:

                    def gb(g, _, lb=lb):
                        hs = []
                        for u in range(KE):
                            off = _al(lb + (g * KE + u) * SG)
                            hs.append(pltpu.async_copy(
                                bsrc_hbm.at[pl.ds(off, SG)], isrc[0][u], sem))
                            hs.append(pltpu.async_copy(
                                bdst_hbm.at[pl.ds(off, SG)], idst[0][u], sem))
                        for h in hs:
                            h.wait()
                        gs = [pltpu.async_copy(
                            gfeat.at[plsc.Indices(isrc[0][u],
                                                  ignored_value=-1)],
                            rows[0][u], sem) for u in range(KE)]
                        for h in gs:
                            h.wait()
                        ss = [pltpu.async_copy(
                            rows[0][u],
                            acc.at[plsc.Indices(idst[0][u],
                                                ignored_value=-1)],
                            sem, add=True) for u in range(KE)]
                        for h in ss:
                            h.wait()
                        return 0

                    lax.fori_loop(0, full, gb, 0)
                for u in range(KE - 1):

                    @pl.when(rem > u)
                    def _(u=u, lb=lb, full=full):
                        off = _al(lb + (full * KE + u) * SG)
                        pltpu.sync_copy(bsrc_hbm.at[pl.ds(off, SG)],
                                        isrc[0][u])
                        pltpu.sync_copy(bdst_hbm.at[pl.ds(off, SG)],
                                        idst[0][u])
                        pltpu.sync_copy(
                            gfeat.at[plsc.Indices(isrc[0][u],
                                                  ignored_value=-1)],
                            rows[0][u])
                        pltpu.sync_copy(
                            rows[0][u],
                            acc.at[plsc.Indices(idst[0][u],
                                                ignored_value=-1)],
                            add=True)
            plsc.subcore_barrier()
            NF = R * KE
            for j0 in range(0, n_full, NF):
                js = range(j0, min(j0 + NF, n_full))
                hs = [pltpu.async_copy(
                    acc.at[pl.ds(_al(off0 + j * SG), SG), :],
                    rowsf[j - j0], sem) for j in js]
                for h in hs:
                    h.wait()
                hs = [pltpu.async_copy(
                    rowsf[j - j0],
                    out_hbm.at[pl.ds(_al(b * span_pad + off0 + j * SG), SG), :],
                    sem) for j in js]
                for h in hs:
                    h.wait()
            if tl:
                pltpu.sync_copy(acc.at[pl.ds(_al(off0 + n_full * SG), tl), :],
                                z0.at[pl.ds(0, tl), :])
                pltpu.sync_copy(
                    z0.at[pl.ds(0, tl), :],
                    out_hbm.at[pl.ds(_al(b * span_pad + off0 + n_full * SG), tl), :])

    return pl.kernel(
        body,
        out_type=jax.ShapeDtypeStruct((NB * span_pad, D), jnp.float32),
        mesh=_mesh(),
        compiler_params=pltpu.CompilerParams(
            needs_layout_passes=False, use_tc_tiling_on_sc=False),
        scratch_types=(
            [pltpu.VMEM_SHARED((span_pad, D), jnp.float32)]
            + [pltpu.VMEM((SG,), jnp.int32) for _ in range(2 * R * KE)]
            + [pltpu.VMEM((SG, D), jnp.float32) for _ in range(R * KE)]
            + [pltpu.VMEM((LN,), jnp.int32)]
            + [pltpu.SemaphoreType.DMA] * 3
        ),
    )


def _unpad(x_pad, NB, span_pad, span, n):
    parts = [x_pad[b * span_pad:b * span_pad + span] for b in range(NB)]
    return jnp.concatenate(parts, axis=0)[:n]


def _l2norm(x):
    norm = jnp.sqrt(jnp.sum(x * x, axis=1, keepdims=True))
    return x / jnp.maximum(norm, 1e-12)


def _graph_setup(e_rows, e_cols, E, nA, N, span, NB, symmetric):
    CH = _round_up((E + NW - 1) // NW, 8)
    if symmetric:
        cap_edges = max(
            2 * CH if any(
                b * span < nA < (b + 1) * span for b in range(NB)) else CH,
            CH)
    else:
        cap_edges = CH
    CAP = _round_up(cap_edges + SG, SG)
    span_pad = _round_up(span, SG)

    pad = NW * CH - E
    if pad:
        fill = jnp.full((pad,), -(nA + 1), jnp.int32)
        e_rows = jnp.concatenate([e_rows, fill])
        e_cols = jnp.concatenate([e_cols, fill])

    bk = _make_bucketize(CH, nA, N, span, NB, CAP, symmetric)
    bsrc, bdst, counts = bk(e_rows, e_cols)
    deg_pad = _make_degree(NB, span_pad, CAP)(bdst, counts)
    deg = _unpad(deg_pad, NB, span_pad, span, N)
    return (bsrc, bdst, counts, CAP, span_pad), deg


def _spmm(gfeat, lists, NB, span_pad, span, N):
    bsrc, bdst, counts, CAP, _ = lists
    out_pad = _make_spmm(NB, span_pad, CAP)(gfeat, bsrc, bdst, counts)
    parts = [out_pad[b * span_pad:b * span_pad + span] for b in range(NB)]
    return jnp.concatenate(parts, axis=0)[:N]


def _propagate(A_feat, B_feat, e_rows, e_cols, E, span, NB, num_layers):
    nA = A_feat.shape[0]
    N = nA + B_feat.shape[0]
    lists, deg = _graph_setup(e_rows, e_cols, E, nA, N, span, NB, True)
    span_pad = lists[4]
    dinv = (1.0 / (jnp.sqrt(deg) + 1e-8))[:, None]

    features = jnp.concatenate([A_feat, B_feat], axis=0)
    total = features
    for i in range(num_layers):
        g = features * dinv
        ssum = _spmm(g, lists, NB, span_pad, span, N)
        features = ssum * dinv / (i + 2)
        total = total + _l2norm(features)
    return total[:nA], total[nA:]


def kernel(users_feat, bundles_feat, items_feat, ui_edges, ub_edges,
           bi_edges):
    IL_users, IL_items = _propagate(
        users_feat, items_feat, ui_edges[0], ui_edges[1],
        E=ui_edges.shape[1], span=25000, NB=4, num_layers=2)

    lists_bi, size = _graph_setup(
        bi_edges[0], bi_edges[1], bi_edges.shape[1], nA=0, N=B,
        span=10000, NB=2, symmetric=False)
    span_pad_bi = lists_bi[4]
    ssum = _spmm(IL_items, lists_bi, 2, span_pad_bi, 10000, B)
    IL_bundles = ssum / (size + 1e-8)[:, None]

    BL_users, BL_bundles = _propagate(
        users_feat, bundles_feat, ub_edges[0], ub_edges[1],
        E=ub_edges.shape[1], span=17500, NB=4, num_layers=2)

    users_out = jnp.concatenate([IL_users, BL_users], axis=1)
    bundles_out = jnp.concatenate([IL_bundles, BL_bundles], axis=1)
    return jnp.concatenate([users_out, bundles_out], axis=0)


# speedup vs baseline: 26.1487x; 1.0582x over previous
import functools

import jax
import jax.numpy as jnp
from jax import lax
from jax.experimental import pallas as pl
from jax.experimental.pallas import tpu as pltpu
from jax.experimental.pallas import tpu_sc as plsc

U = 50000
B = 20000
I = 50000
D = 64

NC = 2
NS = 16
NW = NC * NS
LN = 16
SA = 2048
FLUSH = 1024
STAGE = FLUSH + 32
SG = 128
K = 4


def _mesh():
    return plsc.VectorSubcoreMesh(core_axis_name="c", subcore_axis_name="s")


def _worker_id():
    return lax.axis_index("s") * NC + lax.axis_index("c")


def _round_up(x, m):
    return (x + m - 1) // m * m


def _al(x):
    return pl.multiple_of(x, 8)


@functools.lru_cache(maxsize=None)
def _make_bucketize(CH, nA, N, span, NB, CAP, symmetric):
    n_full = CH // SA
    tail = CH % SA

    def dir_buckets(lo, hi):
        return [b for b in range(NB) if b * span < hi and (b + 1) * span > lo]

    if symmetric:
        dir_cfg = [(0, dir_buckets(0, nA)), (1, dir_buckets(nA, N))]
    else:
        dir_cfg = [(0, dir_buckets(0, N))]

    def body(rows_hbm, cols_hbm, bsrc_hbm, bdst_hbm, counts_hbm,
             rows_v, cols_v, stg_src, stg_dst, cvec_v):
        w = _worker_id()
        base = w * CH
        lanes = lax.iota(jnp.int32, LN)

        def append(b, cnt, off, src_vals, dst_vals, m):
            sb = b * STAGE
            inc = jnp.where(m, 1, 0)
            excl = plsc.cumsum(inc) - inc
            idx = jnp.where(m, sb + cnt + excl, NB * STAGE)
            plsc.store_scatter(stg_src, [idx], src_vals)
            plsc.store_scatter(stg_dst, [idx], dst_vals)
            cnt = cnt + jnp.sum(inc)

            def do_flush(args):
                cnt, off = args
                lb = (w * NB + b) * CAP
                pltpu.sync_copy(stg_src.at[pl.ds(sb, FLUSH)],
                                bsrc_hbm.at[pl.ds(_al(lb + off), FLUSH)])
                pltpu.sync_copy(stg_dst.at[pl.ds(sb, FLUSH)],
                                bdst_hbm.at[pl.ds(_al(lb + off), FLUSH)])
                rs = stg_src[pl.ds(sb + FLUSH, LN)]
                rd = stg_dst[pl.ds(sb + FLUSH, LN)]
                stg_src[pl.ds(sb, LN)] = rs
                stg_dst[pl.ds(sb, LN)] = rd
                return cnt - FLUSH, off + FLUSH

            return lax.cond(cnt >= FLUSH, do_flush, lambda a: a, (cnt, off))

        def proc_vreg(r, c, m_valid, state):
            outs = list(state)
            if symmetric:
                cpn = c + nA
                pairs = [(r, cpn, dir_cfg[0][1]), (cpn, r, dir_cfg[1][1])]
            else:
                pairs = [(r, c, dir_cfg[0][1])]
            for dval, sval, blist in pairs:
                for b in blist:
                    m = (dval >= b * span) & (dval < (b + 1) * span)
                    if m_valid is not None:
                        m = m & m_valid
                    cnt, off = outs[b]
                    outs[b] = append(b, cnt, off, sval, dval - b * span, m)
            return tuple(outs)

        def run_chunk(buf_len, state):
            nv = buf_len // LN
            rem = buf_len % LN

            def vbody(v, st):
                r = rows_v[pl.ds(v * LN, LN)]
                c = cols_v[pl.ds(v * LN, LN)]
                return proc_vreg(r, c, None, st)

            state = lax.fori_loop(0, nv, vbody, state)
            if rem:
                r = rows_v[pl.ds(nv * LN, LN)]
                c = cols_v[pl.ds(nv * LN, LN)]
                state = proc_vreg(r, c, lanes < rem, state)
            return state

        state = tuple((jnp.int32(0), jnp.int32(0)) for _ in range(NB))

        def outer(i, st):
            pltpu.sync_copy(rows_hbm.at[pl.ds(_al(base + i * SA), SA)], rows_v)
            pltpu.sync_copy(cols_hbm.at[pl.ds(_al(base + i * SA), SA)], cols_v)
            return run_chunk(SA, st)

        state = lax.fori_loop(0, n_full, outer, state)
        if tail:
            pltpu.sync_copy(rows_hbm.at[pl.ds(_al(base + n_full * SA), tail)],
                            rows_v.at[pl.ds(0, tail)])
            pltpu.sync_copy(cols_hbm.at[pl.ds(_al(base + n_full * SA), tail)],
                            cols_v.at[pl.ds(0, tail)])
            state = run_chunk(tail, state)

        cnt_vec = jnp.zeros((LN,), jnp.int32)
        for b in range(NB):
            cnt, off = state[b]
            padded = (cnt + SG - 1) // SG * SG
            neg1 = jnp.full((LN,), -1, jnp.int32)
            sb = b * STAGE
            for k in range(SG // LN):
                pos = sb + cnt + k * LN

                @pl.when(cnt + k * LN < padded)
                def _(pos=pos):
                    stg_src[pl.ds(pos, LN)] = neg1
                    stg_dst[pl.ds(pos, LN)] = neg1

            for j in range(FLUSH // SG):

                @pl.when(j * SG < padded)
                def _(b=b, j=j, off=off, sb=sb):
                    lb = (w * NB + b) * CAP
                    pltpu.sync_copy(
                        stg_src.at[pl.ds(sb + j * SG, SG)],
                        bsrc_hbm.at[pl.ds(_al(lb + off + j * SG), SG)])
                    pltpu.sync_copy(
                        stg_dst.at[pl.ds(sb + j * SG, SG)],
                        bdst_hbm.at[pl.ds(_al(lb + off + j * SG), SG)])

            trips = (off + padded) // SG
            cnt_vec = jnp.where(lanes == b, trips, cnt_vec)
        cvec_v[...] = cnt_vec
        pltpu.sync_copy(cvec_v, counts_hbm.at[pl.ds(_al(w * LN), LN)])

    return pl.kernel(
        body,
        out_type=(
            jax.ShapeDtypeStruct((NW * NB * CAP,), jnp.int32),
            jax.ShapeDtypeStruct((NW * NB * CAP,), jnp.int32),
            jax.ShapeDtypeStruct((NW * LN,), jnp.int32),
        ),
        mesh=_mesh(),
        compiler_params=pltpu.CompilerParams(
            needs_layout_passes=False, use_tc_tiling_on_sc=False),
        scratch_types=[
            pltpu.VMEM((SA,), jnp.int32),
            pltpu.VMEM((SA,), jnp.int32),
            pltpu.VMEM((NB * STAGE + LN,), jnp.int32),
            pltpu.VMEM((NB * STAGE + LN,), jnp.int32),
            pltpu.VMEM((LN,), jnp.int32),
        ],
    )


@functools.lru_cache(maxsize=None)
def _make_degree(NB, span_pad, CAP):
    TPT = span_pad // NS
    n_full = TPT // SG
    tl = TPT % SG
    rounds = NB // NC

    def body(bdst_hbm, counts_hbm, deg_hbm, acc, idst0, idst1, idst2, idst3,
             ones_v, zeros_v, bounce_v, cvec_v, sem):
        c = lax.axis_index("c")
        s = lax.axis_index("s")
        lanes = lax.iota(jnp.int32, LN)
        idst = [idst0, idst1, idst2, idst3]

        def ib(i, _):
            ones_v[pl.ds(i * LN, LN)] = jnp.full((LN,), 1.0, jnp.float32)
            zeros_v[pl.ds(i * LN, LN)] = jnp.zeros((LN,), jnp.float32)
            return 0

        lax.fori_loop(0, SG // LN, ib, 0)

        off0 = s * TPT
        for r in range(rounds):
            b = r * NC + c
            for j0 in range(0, n_full, K):
                hs = [pltpu.async_copy(
                    zeros_v, acc.at[pl.ds(_al(off0 + j * SG), SG)], sem)
                    for j in range(j0, min(j0 + K, n_full))]
                for h in hs:
                    h.wait()
            if tl:
                pltpu.sync_copy(zeros_v.at[pl.ds(0, tl)],
                                acc.at[pl.ds(_al(off0 + n_full * SG), tl)])
            plsc.subcore_barrier()
            for wk in range(NW // NS):
                w = s + NS * wk
                pltpu.sync_copy(counts_hbm.at[pl.ds(_al(w * LN), LN)], cvec_v)
                cv = cvec_v[...]
                trips = jnp.max(jnp.where(lanes == b, cv, 0))
                full = trips // K
                rem = trips - full * K
                lb = (w * NB + b) * CAP

                def gb(g, _, lb=lb):
                    hs = [pltpu.async_copy(
                        bdst_hbm.at[pl.ds(_al(lb + (g * K + u) * SG), SG)],
                        idst[u], sem) for u in range(K)]
                    for h in hs:
                        h.wait()
                    ss = [pltpu.async_copy(
                        ones_v,
                        acc.at[plsc.Indices(idst[u], ignored_value=-1)],
                        sem, add=True) for u in range(K)]
                    for h in ss:
                        h.wait()
                    return 0

                lax.fori_loop(0, full, gb, 0)
                for u in range(K - 1):

                    @pl.when(rem > u)
                    def _(u=u, lb=lb, full=full):
                        pltpu.sync_copy(
                            bdst_hbm.at[pl.ds(_al(lb + (full * K + u) * SG),
                                              SG)],
                            idst[u])
                        pltpu.sync_copy(
                            ones_v,
                            acc.at[plsc.Indices(idst[u], ignored_value=-1)],
                            add=True)
            plsc.subcore_barrier()
            for j in range(n_full):
                pltpu.sync_copy(acc.at[pl.ds(_al(off0 + j * SG), SG)],
                                bounce_v)
                pltpu.sync_copy(
                    bounce_v,
                    deg_hbm.at[pl.ds(_al(b * span_pad + off0 + j * SG), SG)])
            if tl:
                pltpu.sync_copy(acc.at[pl.ds(_al(off0 + n_full * SG), tl)],
                                bounce_v.at[pl.ds(0, tl)])
                pltpu.sync_copy(
                    bounce_v.at[pl.ds(0, tl)],
                    deg_hbm.at[pl.ds(_al(b * span_pad + off0 + n_full * SG), tl)])

    return pl.kernel(
        body,
        out_type=jax.ShapeDtypeStruct((NB * span_pad,), jnp.float32),
        mesh=_mesh(),
        compiler_params=pltpu.CompilerParams(
            needs_layout_passes=False, use_tc_tiling_on_sc=False),
        scratch_types=[
            pltpu.VMEM_SHARED((span_pad,), jnp.float32),
            pltpu.VMEM((SG,), jnp.int32),
            pltpu.VMEM((SG,), jnp.int32),
            pltpu.VMEM((SG,), jnp.int32),
            pltpu.VMEM((SG,), jnp.int32),
            pltpu.VMEM((SG,), jnp.float32),
            pltpu.VMEM((SG,), jnp.float32),
            pltpu.VMEM((SG,), jnp.float32),
            pltpu.VMEM((LN,), jnp.int32),
            pltpu.SemaphoreType.DMA,
        ],
    )


@functools.lru_cache(maxsize=None)
def _make_spmm(NB, span_pad, CAP):
    TPT = span_pad // NS
    n_full = TPT // SG
    tl = TPT % SG
    rounds = NB // NC
    per_tile = (2 * 1024 * 1024 - 1 - span_pad * D) // NS
    KE2 = min(K, (per_tile - 4096) // (2 * SG * D))
    if KE2 >= 2:
        R, KE = 2, KE2
    else:
        R, KE = 1, max(1, min(K, (per_tile - 2048) // (SG * D)))

    def body(*refs):
        gfeat, bsrc_hbm, bdst_hbm, counts_hbm, out_hbm, acc = refs[:6]
        flat = list(refs[6:6 + 3 * R * KE])
        isrc = [flat[p * KE:(p + 1) * KE] for p in range(R)]
        idst = [flat[(R + p) * KE:(R + p + 1) * KE] for p in range(R)]
        rows = [flat[(2 * R + p) * KE:(2 * R + p + 1) * KE]
                for p in range(R)]
        cvec_v = refs[6 + 3 * R * KE]
        sem = refs[6 + 3 * R * KE + 1]
        sem_i = refs[6 + 3 * R * KE + 2]
        sem_s = refs[6 + 3 * R * KE + 3]
        c = lax.axis_index("c")
        s = lax.axis_index("s")
        lanes = lax.iota(jnp.int32, LN)

        off0 = s * TPT
        z0 = rows[0][0]
        rowsf = [rw for p in range(R) for rw in rows[p]]
        for r in range(rounds):
            b = r * NC + c

            def zb(i, _):
                for k in range(D // LN):
                    z0[i, pl.ds(k * LN, LN)] = jnp.zeros((LN,), jnp.float32)
                return 0

            lax.fori_loop(0, SG, zb, 0)
            for j0 in range(0, n_full, K):
                hs = [pltpu.async_copy(
                    z0, acc.at[pl.ds(_al(off0 + j * SG), SG), :], sem)
                    for j in range(j0, min(j0 + K, n_full))]
                for h in hs:
                    h.wait()
            if tl:
                pltpu.sync_copy(z0.at[pl.ds(0, tl), :],
                                acc.at[pl.ds(_al(off0 + n_full * SG), tl), :])
            plsc.subcore_barrier()
            for wk in range(NW // NS):
                w = s + NS * wk
                pltpu.sync_copy(counts_hbm.at[pl.ds(_al(w * LN), LN)], cvec_v)
                cv = cvec_v[...]
                trips = jnp.max(jnp.where(lanes == b, cv, 0))
                full = trips // KE
                rem = trips - full * KE
                lb = (w * NB + b) * CAP

                def fire_idx(g, p, lb=lb):
                    for u in range(KE):
                        off = _al(lb + (g * KE + u) * SG)
                        pltpu.async_copy(bsrc_hbm.at[pl.ds(off, SG)],
                                         isrc[p][u], sem_i)
                        pltpu.async_copy(bdst_hbm.at[pl.ds(off, SG)],
                                         idst[p][u], sem_i)

                def drain_idx(lb=lb):
                    for u in range(KE):
                        pltpu.make_async_copy(bsrc_hbm.at[pl.ds(_al(lb), SG)],
                                              isrc[0][u], sem_i).wait()
                        pltpu.make_async_copy(bsrc_hbm.at[pl.ds(_al(lb), SG)],
                                              idst[0][u], sem_i).wait()

                def drain_sc():
                    for u in range(KE):
                        pltpu.make_async_copy(gfeat.at[pl.ds(0, SG), :],
                                              rows[0][u], sem_s).wait()

                if R == 2:

                    def sub(g, p):
                        q = 1 - p
                        drain_idx()
                        gs = [pltpu.async_copy(
                            gfeat.at[plsc.Indices(isrc[p][u],
                                                  ignored_value=-1)],
                            rows[p][u], sem) for u in range(KE)]

                        @pl.when(g > 0)
                        def _():
                            drain_sc()

                        fire_idx(g + 1, q)
                        for h in gs:
                            h.wait()
                        for u in range(KE):
                            pltpu.async_copy(
                                rows[p][u],
                                acc.at[plsc.Indices(idst[p][u],
                                                    ignored_value=-1)],
                                sem_s, add=True)

                    fire_idx(jnp.int32(0), 0)

                    def gb2(h2, _):
                        sub(2 * h2, 0)
                        sub(2 * h2 + 1, 1)
                        return 0

                    full2 = full // 2
                    lax.fori_loop(0, full2, gb2, 0)

                    @pl.when(full - full2 * 2 > 0)
                    def _(full=full):
                        sub(full - 1, 0)

                    drain_idx()

                    @pl.when(full > 0)
                    def _():
                        drain_sc()
                else:

                    def gb(g, _, lb=lb):
                        hs = []
                        for u in range(KE):
                            off = _al(lb + (g * KE + u) * SG)
                            hs.append(pltpu.async_copy(
                                bsrc_hbm.at[pl.ds(off, SG)], isrc[0][u], sem))
                            hs.append(pltpu.async_copy(
                                bdst_hbm.at[pl.ds(off, SG)], idst[0][u], sem))
                        for h in hs:
                            h.wait()
                        gs = [pltpu.async_copy(
                            gfeat.at[plsc.Indices(isrc[0][u],
                                                  ignored_value=-1)],
                            rows[0][u], sem) for u in range(KE)]
                        for h in gs:
                            h.wait()
                        ss = [pltpu.async_copy(
                            rows[0][u],
                            acc.at[plsc.Indices(idst[0][u],
                                                ignored_value=-1)],
                            sem, add=True) for u in range(KE)]
                        for h in ss:
                            h.wait()
                        return 0

                    lax.fori_loop(0, full, gb, 0)
                for u in range(KE - 1):

                    @pl.when(rem > u)
                    def _(u=u, lb=lb, full=full):
                        off = _al(lb + (full * KE + u) * SG)
                        pltpu.sync_copy(bsrc_hbm.at[pl.ds(off, SG)],
                                        isrc[0][u])
                        pltpu.sync_copy(bdst_hbm.at[pl.ds(off, SG)],
                                        idst[0][u])
                        pltpu.sync_copy(
                            gfeat.at[plsc.Indices(isrc[0][u],
                                                  ignored_value=-1)],
                            rows[0][u])
                        pltpu.sync_copy(
                            rows[0][u],
                            acc.at[plsc.Indices(idst[0][u],
                                                ignored_value=-1)],
                            add=True)
            plsc.subcore_barrier()
            NF = R * KE
            for j0 in range(0, n_full, NF):
                js = range(j0, min(j0 + NF, n_full))
                hs = [pltpu.async_copy(
                    acc.at[pl.ds(_al(off0 + j * SG), SG), :],
                    rowsf[j - j0], sem) for j in js]
                for h in hs:
                    h.wait()
                hs = [pltpu.async_copy(
                    rowsf[j - j0],
                    out_hbm.at[pl.ds(_al(b * span_pad + off0 + j * SG), SG), :],
                    sem) for j in js]
                for h in hs:
                    h.wait()
            if tl:
                pltpu.sync_copy(acc.at[pl.ds(_al(off0 + n_full * SG), tl), :],
                                z0.at[pl.ds(0, tl), :])
                pltpu.sync_copy(
                    z0.at[pl.ds(0, tl), :],
                    out_hbm.at[pl.ds(_al(b * span_pad + off0 + n_full * SG), tl), :])

    return pl.kernel(
        body,
        out_type=jax.ShapeDtypeStruct((NB * span_pad, D), jnp.float32),
        mesh=_mesh(),
        compiler_params=pltpu.CompilerParams(
            needs_layout_passes=False, use_tc_tiling_on_sc=False),
        scratch_types=(
            [pltpu.VMEM_SHARED((span_pad, D), jnp.float32)]
            + [pltpu.VMEM((SG,), jnp.int32) for _ in range(2 * R * KE)]
            + [pltpu.VMEM((SG, D), jnp.float32) for _ in range(R * KE)]
            + [pltpu.VMEM((LN,), jnp.int32)]
            + [pltpu.SemaphoreType.DMA] * 3
        ),
    )


def _unpad(x_pad, NB, span_pad, span, n):
    parts = [x_pad[b * span_pad:b * span_pad + span] for b in range(NB)]
    return jnp.concatenate(parts, axis=0)[:n]


def _l2norm(x):
    norm = jnp.sqrt(jnp.sum(x * x, axis=1, keepdims=True))
    return x / jnp.maximum(norm, 1e-12)


def _graph_setup(e_rows, e_cols, E, nA, N, span, NB, symmetric):
    CH = _round_up((E + NW - 1) // NW, 8)
    if symmetric:
        cap_edges = max(
            2 * CH if any(
                b * span < nA < (b + 1) * span for b in range(NB)) else CH,
            CH)
    else:
        cap_edges = CH
    CAP = _round_up(cap_edges + SG, SG)
    span_pad = _round_up(span, SG)

    pad = NW * CH - E
    if pad:
        fill = jnp.full((pad,), -(nA + 1), jnp.int32)
        e_rows = jnp.concatenate([e_rows, fill])
        e_cols = jnp.concatenate([e_cols, fill])

    bk = _make_bucketize(CH, nA, N, span, NB, CAP, symmetric)
    bsrc, bdst, counts = bk(e_rows, e_cols)
    deg_pad = _make_degree(NB, span_pad, CAP)(bdst, counts)
    deg = _unpad(deg_pad, NB, span_pad, span, N)
    return (bsrc, bdst, counts, CAP, span_pad), deg


def _spmm(gfeat, lists, NB, span_pad, span, N):
    bsrc, bdst, counts, CAP, _ = lists
    out_pad = _make_spmm(NB, span_pad, CAP)(gfeat, bsrc, bdst, counts)
    parts = [out_pad[b * span_pad:b * span_pad + span] for b in range(NB)]
    return jnp.concatenate(parts, axis=0)[:N]


def _propagate(A_feat, B_feat, e_rows, e_cols, E, span, NB, num_layers):
    nA = A_feat.shape[0]
    N = nA + B_feat.shape[0]
    lists, deg = _graph_setup(e_rows, e_cols, E, nA, N, span, NB, True)
    span_pad = lists[4]
    dinv = (1.0 / (jnp.sqrt(deg) + 1e-8))[:, None]

    features = jnp.concatenate([A_feat, B_feat], axis=0)
    total = features
    for i in range(num_layers):
        g = features * dinv
        ssum = _spmm(g, lists, NB, span_pad, span, N)
        features = ssum * dinv / (i + 2)
        total = total + _l2norm(features)
    return total[:nA], total[nA:]


def kernel(users_feat, bundles_feat, items_feat, ui_edges, ub_edges,
           bi_edges):
    IL_users, IL_items = _propagate(
        users_feat, items_feat, ui_edges[0], ui_edges[1],
        E=ui_edges.shape[1], span=12500, NB=8, num_layers=2)

    lists_bi, size = _graph_setup(
        bi_edges[0], bi_edges[1], bi_edges.shape[1], nA=0, N=B,
        span=10000, NB=2, symmetric=False)
    span_pad_bi = lists_bi[4]
    ssum = _spmm(IL_items, lists_bi, 2, span_pad_bi, 10000, B)
    IL_bundles = ssum / (size + 1e-8)[:, None]

    BL_users, BL_bundles = _propagate(
        users_feat, bundles_feat, ub_edges[0], ub_edges[1],
        E=ub_edges.shape[1], span=17500, NB=4, num_layers=2)

    users_out = jnp.concatenate([IL_users, BL_users], axis=1)
    bundles_out = jnp.concatenate([IL_bundles, BL_bundles], axis=1)
    return jnp.concatenate([users_out, bundles_out], axis=0)
